# baseline (speedup 1.0000x reference)
import functools

import jax
import jax.numpy as jnp
from jax import lax
from jax.experimental import pallas as pl
from jax.experimental.pallas import tpu as pltpu
from jax.experimental.pallas import tpu_sc as plsc

N_UP = 16384
N_DOWN = 4096
UP_C = 256
DOWN_C = 512
OUT_C = 256

BU = 512
NSPLIT = 8
NH = N_UP // NSPLIT
NW = 32
PW = NH // NW
P = 32
CH = PW // P
GP = P * 3


def _down_proj_body(x_ref, w_ref, b_ref, o_ref):
    o_ref[...] = (
        jax.lax.dot_general(
            x_ref[...].astype(jnp.bfloat16), w_ref[...].astype(jnp.bfloat16),
            (((1,), (0,)), ((), ())),
            preferred_element_type=jnp.float32,
        )
        + b_ref[...]
    )


def _knn_body(up_ref, upf_ref, dpt_ref, wup_ref, bup_ref,
              idx_ref, dense_ref):
    up = up_ref[...]
    dpt = dpt_ref[...]
    up2 = jnp.sum(up * up, axis=1, keepdims=True)
    dp2 = jnp.sum(dpt * dpt, axis=0, keepdims=True)
    cross = jax.lax.dot_general(
        up.astype(jnp.bfloat16), dpt.astype(jnp.bfloat16),
        (((1,), (0,)), ((), ())),
        preferred_element_type=jnp.float32,
    )
    d2 = up2 + dp2 - 2.0 * cross

    iota = jax.lax.broadcasted_iota(jnp.int32, (1, N_DOWN), 1)
    bits = jax.lax.bitcast_convert_type(d2, jnp.int32)
    key = bits ^ (jax.lax.shift_right_arithmetic(bits, 31) & jnp.int32(0x7FFFFFFF))
    kp = (key & jnp.int32(-4096)) | iota
    idxs = []
    vals = []
    for j in range(3):
        m = jnp.min(kp, axis=1, keepdims=True)
        idxs.append(m & jnp.int32(0xFFF))
        kt = m & jnp.int32(-4096)
        d2t = jax.lax.bitcast_convert_type(
            kt ^ (jax.lax.shift_right_arithmetic(kt, 31) & jnp.int32(0x7FFFFFFF)),
            jnp.float32)
        vals.append(d2t)
        if j < 2:
            kp = jnp.where(kp == m, jnp.int32(0x7FFFFFFF), kp)

    ws = [1.0 / (jnp.maximum(v, 0.0) + 1e-8) for v in vals]
    wsum = ws[0] + ws[1] + ws[2]
    ws = [w / wsum for w in ws]

    idx_ref[...] = jnp.concatenate(idxs, axis=1)

    dense = (
        jax.lax.dot_general(
            upf_ref[...].astype(jnp.bfloat16), wup_ref[...].astype(jnp.bfloat16),
            (((1,), (0,)), ((), ())),
            preferred_element_type=jnp.float32,
        )
        + bup_ref[...]
    )
    dense_ref[...] = jnp.concatenate(
        [dense] + [jnp.broadcast_to(w, (BU, 16)) for w in ws], axis=1)


def _sc_interp(downf_hbm, idx_hbm, dense_hbm, out_hbm,
               idx_v, st_v, rows_v0, rows_v1, acc_v, sem0, sem1):
    wid = lax.axis_index("s") * 2 + lax.axis_index("c")
    base = wid * PW
    pltpu.sync_copy(idx_hbm.at[pl.ds(base * 3, PW * 3)], idx_v)

    pltpu.async_copy(downf_hbm.at[idx_v.at[pl.ds(0, GP)]], rows_v0, sem0)
    pltpu.async_copy(downf_hbm.at[idx_v.at[pl.ds(GP, GP)]], rows_v1, sem1)

    def pair_body(g, carry):
        def half(par, rows_b, sem_b):
            c = g * 2 + par
            row0 = base + c * P
            pltpu.make_async_copy(
                downf_hbm.at[idx_v.at[pl.ds(0, GP)]], rows_b, sem_b).wait()
            pltpu.sync_copy(dense_hbm.at[pl.ds(row0, P)], st_v)

            @plsc.parallel_loop(0, P, unroll=8)
            def _(i):
                w0 = st_v[i, pl.ds(OUT_C, 16)]
                w1 = st_v[i, pl.ds(OUT_C + 16, 16)]
                w2 = st_v[i, pl.ds(OUT_C + 32, 16)]
                r3 = i * 3
                for s in range(OUT_C // 16):
                    sl = pl.ds(s * 16, 16)
                    seg = st_v[i, sl]
                    seg = seg + w0 * rows_b[r3, sl]
                    seg = seg + w1 * rows_b[r3 + 1, sl]
                    seg = seg + w2 * rows_b[r3 + 2, sl]
                    acc_v[i, sl] = seg

            pltpu.sync_copy(acc_v, out_hbm.at[pl.ds(row0, P)])
            nxt = c + 2

            @pl.when(nxt < CH)
            def _():
                pltpu.async_copy(
                    downf_hbm.at[idx_v.at[pl.ds(nxt * GP, GP)]],
                    rows_b, sem_b)

        half(0, rows_v0, sem0)
        half(1, rows_v1, sem1)
        return carry

    lax.fori_loop(0, CH // 2, pair_body, 0)


@functools.lru_cache(maxsize=1)
def _get_sc_interp():
    return functools.partial(
        pl.kernel,
        mesh=plsc.VectorSubcoreMesh(core_axis_name="c", subcore_axis_name="s"),
        out_type=jax.ShapeDtypeStruct((NH, OUT_C), jnp.float32),
        scratch_types=[
            pltpu.VMEM((PW * 3,), jnp.int32),
            pltpu.VMEM((P, OUT_C + 48), jnp.float32),
            pltpu.VMEM((GP, OUT_C), jnp.float32),
            pltpu.VMEM((GP, OUT_C), jnp.float32),
            pltpu.VMEM((P, OUT_C), jnp.float32),
            pltpu.SemaphoreType.DMA,
            pltpu.SemaphoreType.DMA,
        ],
    )(_sc_interp)


@jax.jit
def kernel(up_points, up_features, down_points, down_features, W_up, b_up, W_down, b_down):
    b_down2 = b_down[None, :]
    b_up2 = b_up[None, :]

    down_f = pl.pallas_call(
        _down_proj_body,
        grid=(4,),
        in_specs=[
            pl.BlockSpec((N_DOWN // 4, DOWN_C), lambda i: (i, 0)),
            pl.BlockSpec((DOWN_C, OUT_C), lambda i: (0, 0)),
            pl.BlockSpec((1, OUT_C), lambda i: (0, 0)),
        ],
        out_specs=pl.BlockSpec((N_DOWN // 4, OUT_C), lambda i: (i, 0)),
        out_shape=jax.ShapeDtypeStruct((N_DOWN, OUT_C), jnp.float32),
    )(down_features, W_down, b_down2)

    dpt = down_points.T

    outs = []
    for h in range(NSPLIT):
        off = h * (NH // BU)
        idx, dense_ext = pl.pallas_call(
            _knn_body,
            grid=(NH // BU,),
            in_specs=[
                pl.BlockSpec((BU, 3), lambda i, off=off: (i + off, 0)),
                pl.BlockSpec((BU, UP_C), lambda i, off=off: (i + off, 0)),
                pl.BlockSpec((3, N_DOWN), lambda i: (0, 0)),
                pl.BlockSpec((UP_C, OUT_C), lambda i: (0, 0)),
                pl.BlockSpec((1, OUT_C), lambda i: (0, 0)),
            ],
            out_specs=[
                pl.BlockSpec((BU, 3), lambda i: (i, 0)),
                pl.BlockSpec((BU, OUT_C + 48), lambda i: (i, 0)),
            ],
            out_shape=[
                jax.ShapeDtypeStruct((NH, 3), jnp.int32),
                jax.ShapeDtypeStruct((NH, OUT_C + 48), jnp.float32),
            ],
        )(up_points, up_features, dpt, W_up, b_up2)
        outs.append(_get_sc_interp()(down_f, idx.reshape(-1), dense_ext))

    return jnp.concatenate(outs, axis=0)


# speedup vs baseline: 1.0323x; 1.0323x over previous
import functools

import jax
import jax.numpy as jnp
from jax import lax
from jax.experimental import pallas as pl
from jax.experimental.pallas import tpu as pltpu
from jax.experimental.pallas import tpu_sc as plsc

N_UP = 16384
N_DOWN = 4096
UP_C = 256
DOWN_C = 512
OUT_C = 256

BU = 512
NSPLIT = 4
NH = N_UP // NSPLIT
NW = 32
PW = NH // NW
P = 32
CH = PW // P
GP = P * 3


def _down_proj_body(x_ref, w_ref, b_ref, o_ref):
    o_ref[...] = (
        jax.lax.dot_general(
            x_ref[...].astype(jnp.bfloat16), w_ref[...].astype(jnp.bfloat16),
            (((1,), (0,)), ((), ())),
            preferred_element_type=jnp.float32,
        )
        + b_ref[...]
    )


def _knn_body(up_ref, upf_ref, dpt_ref, wup_ref, bup_ref,
              idx_ref, dense_ref):
    up = up_ref[...]
    dpt = dpt_ref[...]
    up2 = jnp.sum(up * up, axis=1, keepdims=True)
    dp2 = jnp.sum(dpt * dpt, axis=0, keepdims=True)
    cross = jax.lax.dot_general(
        up.astype(jnp.bfloat16), dpt.astype(jnp.bfloat16),
        (((1,), (0,)), ((), ())),
        preferred_element_type=jnp.float32,
    )
    d2 = up2 + dp2 - 2.0 * cross

    iota = jax.lax.broadcasted_iota(jnp.int32, (1, N_DOWN), 1)
    bits = jax.lax.bitcast_convert_type(d2, jnp.int32)
    key = bits ^ (jax.lax.shift_right_arithmetic(bits, 31) & jnp.int32(0x7FFFFFFF))
    kp = (key & jnp.int32(-4096)) | iota

    m1 = jax.lax.slice_in_dim(kp, 0, 128, axis=1)
    m2 = jnp.full_like(m1, jnp.int32(0x7FFFFFFF))
    m3 = m2
    for t in range(1, N_DOWN // 128):
        v = jax.lax.slice_in_dim(kp, t * 128, (t + 1) * 128, axis=1)
        hi1 = jnp.maximum(m1, v)
        m1 = jnp.minimum(m1, v)
        hi2 = jnp.maximum(m2, hi1)
        m2 = jnp.minimum(m2, hi1)
        m3 = jnp.minimum(m3, hi2)

    cat = jnp.concatenate([m1, m2, m3], axis=1)
    idxs = []
    vals = []
    for j in range(3):
        m = jnp.min(cat, axis=1, keepdims=True)
        idxs.append(m & jnp.int32(0xFFF))
        kt = m & jnp.int32(-4096)
        d2t = jax.lax.bitcast_convert_type(
            kt ^ (jax.lax.shift_right_arithmetic(kt, 31) & jnp.int32(0x7FFFFFFF)),
            jnp.float32)
        vals.append(d2t)
        if j < 2:
            cat = jnp.where(cat == m, jnp.int32(0x7FFFFFFF), cat)

    ws = [1.0 / (jnp.maximum(v, 0.0) + 1e-8) for v in vals]
    wsum = ws[0] + ws[1] + ws[2]
    ws = [w / wsum for w in ws]

    idx_ref[...] = jnp.concatenate(idxs, axis=1)

    dense = (
        jax.lax.dot_general(
            upf_ref[...].astype(jnp.bfloat16), wup_ref[...].astype(jnp.bfloat16),
            (((1,), (0,)), ((), ())),
            preferred_element_type=jnp.float32,
        )
        + bup_ref[...]
    )
    dense_ref[...] = jnp.concatenate(
        [dense] + [jnp.broadcast_to(w, (BU, 16)) for w in ws], axis=1)


def _sc_interp(downf_hbm, idx_hbm, dense_hbm, out_hbm,
               idx_v, st_v, rows_v0, rows_v1, acc_v, sem0, sem1):
    wid = lax.axis_index("s") * 2 + lax.axis_index("c")
    base = wid * PW
    pltpu.sync_copy(idx_hbm.at[pl.ds(base * 3, PW * 3)], idx_v)

    pltpu.async_copy(downf_hbm.at[idx_v.at[pl.ds(0, GP)]], rows_v0, sem0)
    pltpu.async_copy(downf_hbm.at[idx_v.at[pl.ds(GP, GP)]], rows_v1, sem1)

    def pair_body(g, carry):
        def half(par, rows_b, sem_b):
            c = g * 2 + par
            row0 = base + c * P
            pltpu.make_async_copy(
                downf_hbm.at[idx_v.at[pl.ds(0, GP)]], rows_b, sem_b).wait()
            pltpu.sync_copy(dense_hbm.at[pl.ds(row0, P)], st_v)

            @plsc.parallel_loop(0, P, unroll=8)
            def _(i):
                w0 = st_v[i, pl.ds(OUT_C, 16)]
                w1 = st_v[i, pl.ds(OUT_C + 16, 16)]
                w2 = st_v[i, pl.ds(OUT_C + 32, 16)]
                r3 = i * 3
                for s in range(OUT_C // 16):
                    sl = pl.ds(s * 16, 16)
                    seg = st_v[i, sl]
                    seg = seg + w0 * rows_b[r3, sl]
                    seg = seg + w1 * rows_b[r3 + 1, sl]
                    seg = seg + w2 * rows_b[r3 + 2, sl]
                    acc_v[i, sl] = seg

            pltpu.sync_copy(acc_v, out_hbm.at[pl.ds(row0, P)])
            nxt = c + 2

            @pl.when(nxt < CH)
            def _():
                pltpu.async_copy(
                    downf_hbm.at[idx_v.at[pl.ds(nxt * GP, GP)]],
                    rows_b, sem_b)

        half(0, rows_v0, sem0)
        half(1, rows_v1, sem1)
        return carry

    lax.fori_loop(0, CH // 2, pair_body, 0)


@functools.lru_cache(maxsize=1)
def _get_sc_interp():
    return functools.partial(
        pl.kernel,
        mesh=plsc.VectorSubcoreMesh(core_axis_name="c", subcore_axis_name="s"),
        out_type=jax.ShapeDtypeStruct((NH, OUT_C), jnp.float32),
        scratch_types=[
            pltpu.VMEM((PW * 3,), jnp.int32),
            pltpu.VMEM((P, OUT_C + 48), jnp.float32),
            pltpu.VMEM((GP, OUT_C), jnp.float32),
            pltpu.VMEM((GP, OUT_C), jnp.float32),
            pltpu.VMEM((P, OUT_C), jnp.float32),
            pltpu.SemaphoreType.DMA,
            pltpu.SemaphoreType.DMA,
        ],
    )(_sc_interp)


@jax.jit
def kernel(up_points, up_features, down_points, down_features, W_up, b_up, W_down, b_down):
    b_down2 = b_down[None, :]
    b_up2 = b_up[None, :]

    down_f = pl.pallas_call(
        _down_proj_body,
        grid=(4,),
        in_specs=[
            pl.BlockSpec((N_DOWN // 4, DOWN_C), lambda i: (i, 0)),
            pl.BlockSpec((DOWN_C, OUT_C), lambda i: (0, 0)),
            pl.BlockSpec((1, OUT_C), lambda i: (0, 0)),
        ],
        out_specs=pl.BlockSpec((N_DOWN // 4, OUT_C), lambda i: (i, 0)),
        out_shape=jax.ShapeDtypeStruct((N_DOWN, OUT_C), jnp.float32),
    )(down_features, W_down, b_down2)

    dpt = down_points.T

    outs = []
    for h in range(NSPLIT):
        off = h * (NH // BU)
        idx, dense_ext = pl.pallas_call(
            _knn_body,
            grid=(NH // BU,),
            in_specs=[
                pl.BlockSpec((BU, 3), lambda i, off=off: (i + off, 0)),
                pl.BlockSpec((BU, UP_C), lambda i, off=off: (i + off, 0)),
                pl.BlockSpec((3, N_DOWN), lambda i: (0, 0)),
                pl.BlockSpec((UP_C, OUT_C), lambda i: (0, 0)),
                pl.BlockSpec((1, OUT_C), lambda i: (0, 0)),
            ],
            out_specs=[
                pl.BlockSpec((BU, 3), lambda i: (i, 0)),
                pl.BlockSpec((BU, OUT_C + 48), lambda i: (i, 0)),
            ],
            out_shape=[
                jax.ShapeDtypeStruct((NH, 3), jnp.int32),
                jax.ShapeDtypeStruct((NH, OUT_C + 48), jnp.float32),
            ],
        )(up_points, up_features, dpt, W_up, b_up2)
        outs.append(_get_sc_interp()(down_f, idx.reshape(-1), dense_ext))

    return jnp.concatenate(outs, axis=0)
